# baseline (speedup 1.0000x reference)
import functools
import math

import jax
import jax.numpy as jnp
from jax import lax
from jax.experimental import pallas as pl
from jax.experimental.pallas import tpu as pltpu
from jax.experimental.pallas import tpu_sc as plsc

_KEY0 = 0
_KEY1 = 42
_PARITY = 0x1BD11BDA


def _rotl(x, r):
    return lax.shift_left(x, r) | lax.shift_right_logical(x, 32 - r)


def _threefry2x32_preadded(x0, x1):
    ks0 = jnp.int32(_KEY0)
    ks1 = jnp.int32(_KEY1)
    ks2 = jnp.int32(_KEY0 ^ _KEY1 ^ _PARITY)
    rot_a = (13, 15, 26, 6)
    rot_b = (17, 29, 16, 24)

    def four_rounds(x0, x1, rots):
        for r in rots:
            x0 = x0 + x1
            x1 = _rotl(x1, r)
            x1 = x0 ^ x1
        return x0, x1

    x0, x1 = four_rounds(x0, x1, rot_a)
    x0 = x0 + ks1
    x1 = x1 + ks2 + 1
    x0, x1 = four_rounds(x0, x1, rot_b)
    x0 = x0 + ks2
    x1 = x1 + ks0 + 2
    x0, x1 = four_rounds(x0, x1, rot_a)
    x0 = x0 + ks0
    x1 = x1 + ks1 + 3
    x0, x1 = four_rounds(x0, x1, rot_b)
    x0 = x0 + ks1
    x1 = x1 + ks2 + 4
    x0, x1 = four_rounds(x0, x1, rot_a)
    x0 = x0 + ks2
    x1 = x1 + ks0 + 5
    return x0, x1


def _neg_exp_body(w_ref, o_ref):
    o_ref[...] = -jnp.exp(-w_ref[...])


def _make_neg_exp(B, N, JC, interpret=False):
    nchunk = N // JC
    rb = min(8, B)
    return pl.pallas_call(
        _neg_exp_body,
        out_shape=jax.ShapeDtypeStruct((B, nchunk, JC), jnp.float32),
        grid=(B // rb,),
        in_specs=[pl.BlockSpec((rb, nchunk, JC), lambda i: (i, 0, 0))],
        out_specs=pl.BlockSpec((rb, nchunk, JC), lambda i: (i, 0, 0)),
        interpret=interpret,
    )


def _sampler_body(ne_ref, out_ref, *, N, B, SBLK, JC):
    b = pl.program_id(0)
    k = pl.program_id(1)
    s0 = k * SBLK
    logn = N.bit_length() - 1
    logbn = (N * B).bit_length() - 1
    smask = (1 << (32 - logbn)) - 1
    hi0 = (s0 >> (32 - logbn)) + jnp.int32(_KEY0)
    r_iota = lax.broadcasted_iota(jnp.int32, (SBLK, 1), 0)
    lane = lax.broadcasted_iota(jnp.int32, (1, JC), 1)
    lo_base = (lax.shift_left((s0 & smask) + r_iota, logbn)
               + lax.shift_left(b, logn) + jnp.int32(_KEY1) + lane)
    nchunk = N // JC
    UNROLL = 4

    def chunk(c, carry):
        best, bestc = carry
        for t in range(UNROLL):
            cc = c * UNROLL + t
            lo = lo_base + cc * JC
            o0, o1 = _threefry2x32_preadded(hi0, lo)
            bits = o0 ^ o1
            ub = lax.shift_right_logical(bits, 9) | jnp.int32(0x3F800000)
            u = lax.bitcast_convert_type(ub, jnp.float32) - jnp.float32(1.0)
            ne = ne_ref[0, pl.ds(cc, 1), :]
            score = jnp.log(u) * ne
            m = score < best
            best = jnp.where(m, score, best)
            bestc = jnp.where(m, cc, bestc)
        return best, bestc

    init = (jnp.full((SBLK, JC), jnp.inf, jnp.float32),
            jnp.zeros((SBLK, JC), jnp.int32))
    best, bestc = lax.fori_loop(0, nchunk // UNROLL, chunk, init)
    bestj = lax.shift_left(bestc, JC.bit_length() - 1) + lane
    rowmin = jnp.min(best, axis=1, keepdims=True)
    candj = jnp.where(best == rowmin, bestj, jnp.int32(2 ** 31 - 1))
    idx = jnp.min(candj, axis=1, keepdims=True)
    out_ref[0, :, :] = idx + lax.shift_left(b, logn)


def _make_sampler(B, N, SBLK, JC, interpret=False):
    nchunk = N // JC
    nsb = N // SBLK
    body = functools.partial(_sampler_body, N=N, B=B, SBLK=SBLK, JC=JC)
    return pl.pallas_call(
        body,
        out_shape=jax.ShapeDtypeStruct((B * nsb, SBLK, 1), jnp.int32),
        grid=(B, nsb),
        in_specs=[pl.BlockSpec((1, nchunk, JC), lambda b, k: (b, 0, 0))],
        out_specs=pl.BlockSpec((1, SBLK, 1), lambda b, k: (b * nsb + k, 0, 0)),
        interpret=interpret,
    )


def _gather_body(table_hbm, gidx_hbm, out_hbm, idx_v, rows_v, sem, *,
                 rows_per_w, chunk):
    wid = lax.axis_index("s") * 2 + lax.axis_index("c")
    base = wid * rows_per_w
    nchunk = rows_per_w // chunk

    def step(c, carry):
        off = base + c * chunk
        pltpu.sync_copy(gidx_hbm.at[pl.ds(off, chunk)], idx_v)
        pltpu.async_copy(table_hbm.at[idx_v], rows_v, sem).wait()
        pltpu.sync_copy(rows_v, out_hbm.at[pl.ds(off, chunk)])
        return carry

    lax.fori_loop(0, nchunk, step, 0)


def _make_gather(rows_total, D, chunk=128):
    nw = 32
    rows_per_w = rows_total // nw
    mesh = plsc.VectorSubcoreMesh(core_axis_name="c", subcore_axis_name="s")
    body = functools.partial(_gather_body, rows_per_w=rows_per_w, chunk=chunk)
    return pl.kernel(
        body,
        out_type=jax.ShapeDtypeStruct((rows_total, D), jnp.float32),
        mesh=mesh,
        compiler_params=pltpu.CompilerParams(use_tc_tiling_on_sc=False),
        scratch_types=[
            pltpu.VMEM((chunk,), jnp.int32),
            pltpu.VMEM((chunk, D), jnp.float32),
            pltpu.SemaphoreType.DMA,
        ],
    )


def kernel(state, weight):
    B, N = weight.shape
    D = state.shape[-1]
    JC = 128
    SBLK = 128
    ne = _make_neg_exp(B, N, JC)(weight.reshape(B, N // JC, JC))
    gidx = _make_sampler(B, N, SBLK, JC)(ne).reshape(-1)
    table = state.reshape(B * N, D)
    new_state = _make_gather(B * N, D)(table, gidx).reshape(B, N, D)
    out_weight = jnp.full((B, N), -math.log(N), dtype=weight.dtype)
    return (new_state, out_weight)


# speedup vs baseline: 1.0331x; 1.0331x over previous
import functools
import math

import jax
import jax.numpy as jnp
from jax import lax
from jax.experimental import pallas as pl
from jax.experimental.pallas import tpu as pltpu
from jax.experimental.pallas import tpu_sc as plsc

_KEY0 = 0
_KEY1 = 42
_PARITY = 0x1BD11BDA


def _rotl(x, r):
    return lax.shift_left(x, r) | lax.shift_right_logical(x, 32 - r)


def _threefry2x32_preadded(x0, x1):
    ks0 = jnp.int32(_KEY0)
    ks1 = jnp.int32(_KEY1)
    ks2 = jnp.int32(_KEY0 ^ _KEY1 ^ _PARITY)
    rot_a = (13, 15, 26, 6)
    rot_b = (17, 29, 16, 24)

    def four_rounds(x0, x1, rots):
        for r in rots:
            x0 = x0 + x1
            x1 = _rotl(x1, r)
            x1 = x0 ^ x1
        return x0, x1

    x0, x1 = four_rounds(x0, x1, rot_a)
    x0 = x0 + ks1
    x1 = x1 + ks2 + 1
    x0, x1 = four_rounds(x0, x1, rot_b)
    x0 = x0 + ks2
    x1 = x1 + ks0 + 2
    x0, x1 = four_rounds(x0, x1, rot_a)
    x0 = x0 + ks0
    x1 = x1 + ks1 + 3
    x0, x1 = four_rounds(x0, x1, rot_b)
    x0 = x0 + ks1
    x1 = x1 + ks2 + 4
    x0, x1 = four_rounds(x0, x1, rot_a)
    x0 = x0 + ks2
    x1 = x1 + ks0 + 5
    return x0, x1


def _neg_exp_body(w_ref, o_ref):
    o_ref[...] = -jnp.exp(-w_ref[...])


def _make_neg_exp(B, N, JC, interpret=False):
    nchunk = N // JC
    rb = min(8, B)
    return pl.pallas_call(
        _neg_exp_body,
        out_shape=jax.ShapeDtypeStruct((B, nchunk, JC), jnp.float32),
        grid=(B // rb,),
        in_specs=[pl.BlockSpec((rb, nchunk, JC), lambda i: (i, 0, 0))],
        out_specs=pl.BlockSpec((rb, nchunk, JC), lambda i: (i, 0, 0)),
        interpret=interpret,
    )


def _sampler_body(ne_ref, out_ref, *, N, B, SBLK, JC):
    b = pl.program_id(0)
    k = pl.program_id(1)
    s0 = k * SBLK
    logn = N.bit_length() - 1
    logbn = (N * B).bit_length() - 1
    smask = (1 << (32 - logbn)) - 1
    hi0 = (s0 >> (32 - logbn)) + jnp.int32(_KEY0)
    r_iota = lax.broadcasted_iota(jnp.int32, (SBLK, 1), 0)
    lane = lax.broadcasted_iota(jnp.int32, (1, JC), 1)
    lo_base = (lax.shift_left((s0 & smask) + r_iota, logbn)
               + lax.shift_left(b, logn) + jnp.int32(_KEY1) + lane)
    nchunk = N // JC
    UNROLL = 16

    def chunk(c, carry):
        best, bestc = carry
        for t in range(UNROLL):
            cc = c * UNROLL + t
            lo = lo_base + cc * JC
            o0, o1 = _threefry2x32_preadded(hi0, lo)
            bits = o0 ^ o1
            ub = lax.shift_right_logical(bits, 9) | jnp.int32(0x3F800000)
            u = lax.bitcast_convert_type(ub, jnp.float32) - jnp.float32(1.0)
            ne = ne_ref[0, pl.ds(cc, 1), :]
            score = jnp.log(u) * ne
            m = score < best
            best = jnp.where(m, score, best)
            bestc = jnp.where(m, cc, bestc)
        return best, bestc

    init = (jnp.full((SBLK, JC), jnp.inf, jnp.float32),
            jnp.zeros((SBLK, JC), jnp.int32))
    best, bestc = lax.fori_loop(0, nchunk // UNROLL, chunk, init)
    bestj = lax.shift_left(bestc, JC.bit_length() - 1) + lane
    rowmin = jnp.min(best, axis=1, keepdims=True)
    candj = jnp.where(best == rowmin, bestj, jnp.int32(2 ** 31 - 1))
    idx = jnp.min(candj, axis=1, keepdims=True)
    out_ref[0, :, :] = idx + lax.shift_left(b, logn)


def _make_sampler(B, N, SBLK, JC, interpret=False):
    nchunk = N // JC
    nsb = N // SBLK
    body = functools.partial(_sampler_body, N=N, B=B, SBLK=SBLK, JC=JC)
    return pl.pallas_call(
        body,
        out_shape=jax.ShapeDtypeStruct((B * nsb, SBLK, 1), jnp.int32),
        grid=(B, nsb),
        in_specs=[pl.BlockSpec((1, nchunk, JC), lambda b, k: (b, 0, 0))],
        out_specs=pl.BlockSpec((1, SBLK, 1), lambda b, k: (b * nsb + k, 0, 0)),
        interpret=interpret,
    )


def _gather_body(table_hbm, gidx_hbm, out_hbm, idx_v, rows_v, sem, *,
                 rows_per_w, chunk):
    wid = lax.axis_index("s") * 2 + lax.axis_index("c")
    base = wid * rows_per_w
    nchunk = rows_per_w // chunk

    def step(c, carry):
        off = base + c * chunk
        pltpu.sync_copy(gidx_hbm.at[pl.ds(off, chunk)], idx_v)
        pltpu.async_copy(table_hbm.at[idx_v], rows_v, sem).wait()
        pltpu.sync_copy(rows_v, out_hbm.at[pl.ds(off, chunk)])
        return carry

    lax.fori_loop(0, nchunk, step, 0)


def _make_gather(rows_total, D, chunk=128):
    nw = 32
    rows_per_w = rows_total // nw
    mesh = plsc.VectorSubcoreMesh(core_axis_name="c", subcore_axis_name="s")
    body = functools.partial(_gather_body, rows_per_w=rows_per_w, chunk=chunk)
    return pl.kernel(
        body,
        out_type=jax.ShapeDtypeStruct((rows_total, D), jnp.float32),
        mesh=mesh,
        compiler_params=pltpu.CompilerParams(use_tc_tiling_on_sc=False),
        scratch_types=[
            pltpu.VMEM((chunk,), jnp.int32),
            pltpu.VMEM((chunk, D), jnp.float32),
            pltpu.SemaphoreType.DMA,
        ],
    )


def kernel(state, weight):
    B, N = weight.shape
    D = state.shape[-1]
    JC = 128
    SBLK = 64
    ne = _make_neg_exp(B, N, JC)(weight.reshape(B, N // JC, JC))
    gidx = _make_sampler(B, N, SBLK, JC)(ne).reshape(-1)
    table = state.reshape(B * N, D)
    new_state = _make_gather(B * N, D)(table, gidx).reshape(B, N, D)
    out_weight = jnp.full((B, N), -math.log(N), dtype=weight.dtype)
    return (new_state, out_weight)


# speedup vs baseline: 1.0331x; 1.0000x over previous
import functools
import math

import jax
import jax.numpy as jnp
from jax import lax
from jax.experimental import pallas as pl
from jax.experimental.pallas import tpu as pltpu
from jax.experimental.pallas import tpu_sc as plsc

_KEY0 = 0
_KEY1 = 42
_PARITY = 0x1BD11BDA


def _rotl(x, r):
    return lax.shift_left(x, r) | lax.shift_right_logical(x, 32 - r)


def _threefry2x32_preadded(x0, x1):
    ks0 = jnp.int32(_KEY0)
    ks1 = jnp.int32(_KEY1)
    ks2 = jnp.int32(_KEY0 ^ _KEY1 ^ _PARITY)
    rot_a = (13, 15, 26, 6)
    rot_b = (17, 29, 16, 24)

    def four_rounds(x0, x1, rots):
        for r in rots:
            x0 = x0 + x1
            x1 = _rotl(x1, r)
            x1 = x0 ^ x1
        return x0, x1

    x0, x1 = four_rounds(x0, x1, rot_a)
    x0 = x0 + ks1
    x1 = x1 + ks2 + 1
    x0, x1 = four_rounds(x0, x1, rot_b)
    x0 = x0 + ks2
    x1 = x1 + ks0 + 2
    x0, x1 = four_rounds(x0, x1, rot_a)
    x0 = x0 + ks0
    x1 = x1 + ks1 + 3
    x0, x1 = four_rounds(x0, x1, rot_b)
    x0 = x0 + ks1
    x1 = x1 + ks2 + 4
    x0, x1 = four_rounds(x0, x1, rot_a)
    x0 = x0 + ks2
    x1 = x1 + ks0 + 5
    return x0, x1


def _neg_exp_body(w_ref, o_ref):
    o_ref[...] = -jnp.exp(-w_ref[...])


def _make_neg_exp(B, N, JC, interpret=False):
    nchunk = N // JC
    rb = min(8, B)
    return pl.pallas_call(
        _neg_exp_body,
        out_shape=jax.ShapeDtypeStruct((B, nchunk, JC), jnp.float32),
        grid=(B // rb,),
        in_specs=[pl.BlockSpec((rb, nchunk, JC), lambda i: (i, 0, 0))],
        out_specs=pl.BlockSpec((rb, nchunk, JC), lambda i: (i, 0, 0)),
        interpret=interpret,
    )


def _sampler_body(ne_ref, out_ref, *, N, B, SBLK, JC):
    b = pl.program_id(0)
    k = pl.program_id(1)
    s0 = k * SBLK
    logn = N.bit_length() - 1
    logbn = (N * B).bit_length() - 1
    smask = (1 << (32 - logbn)) - 1
    hi0 = (s0 >> (32 - logbn)) + jnp.int32(_KEY0)
    r_iota = lax.broadcasted_iota(jnp.int32, (SBLK, 1), 0)
    lane = lax.broadcasted_iota(jnp.int32, (1, JC), 1)
    lo_base = (lax.shift_left((s0 & smask) + r_iota, logbn)
               + lax.shift_left(b, logn) + jnp.int32(_KEY1) + lane)
    nchunk = N // JC
    UNROLL = 32

    def chunk(c, carry):
        best, bestc = carry
        for t in range(UNROLL):
            cc = c * UNROLL + t
            lo = lo_base + cc * JC
            o0, o1 = _threefry2x32_preadded(hi0, lo)
            bits = o0 ^ o1
            ub = lax.shift_right_logical(bits, 9) | jnp.int32(0x3F800000)
            u = lax.bitcast_convert_type(ub, jnp.float32) - jnp.float32(1.0)
            ne = ne_ref[0, pl.ds(cc, 1), :]
            score = jnp.log(u) * ne
            m = score < best
            best = jnp.where(m, score, best)
            bestc = jnp.where(m, cc, bestc)
        return best, bestc

    init = (jnp.full((SBLK, JC), jnp.inf, jnp.float32),
            jnp.zeros((SBLK, JC), jnp.int32))
    best, bestc = lax.fori_loop(0, nchunk // UNROLL, chunk, init)
    bestj = lax.shift_left(bestc, JC.bit_length() - 1) + lane
    rowmin = jnp.min(best, axis=1, keepdims=True)
    candj = jnp.where(best == rowmin, bestj, jnp.int32(2 ** 31 - 1))
    idx = jnp.min(candj, axis=1, keepdims=True)
    out_ref[0, :, :] = idx + lax.shift_left(b, logn)


def _make_sampler(B, N, SBLK, JC, interpret=False):
    nchunk = N // JC
    nsb = N // SBLK
    body = functools.partial(_sampler_body, N=N, B=B, SBLK=SBLK, JC=JC)
    return pl.pallas_call(
        body,
        out_shape=jax.ShapeDtypeStruct((B * nsb, SBLK, 1), jnp.int32),
        grid=(B, nsb),
        in_specs=[pl.BlockSpec((1, nchunk, JC), lambda b, k: (b, 0, 0))],
        out_specs=pl.BlockSpec((1, SBLK, 1), lambda b, k: (b * nsb + k, 0, 0)),
        interpret=interpret,
    )


def _gather_body(table_hbm, gidx_hbm, out_hbm, idx_v, rows_v, sem, *,
                 rows_per_w, chunk):
    wid = lax.axis_index("s") * 2 + lax.axis_index("c")
    base = wid * rows_per_w
    nchunk = rows_per_w // chunk

    def step(c, carry):
        off = base + c * chunk
        pltpu.sync_copy(gidx_hbm.at[pl.ds(off, chunk)], idx_v)
        pltpu.async_copy(table_hbm.at[idx_v], rows_v, sem).wait()
        pltpu.sync_copy(rows_v, out_hbm.at[pl.ds(off, chunk)])
        return carry

    lax.fori_loop(0, nchunk, step, 0)


def _make_gather(rows_total, D, chunk=128):
    nw = 32
    rows_per_w = rows_total // nw
    mesh = plsc.VectorSubcoreMesh(core_axis_name="c", subcore_axis_name="s")
    body = functools.partial(_gather_body, rows_per_w=rows_per_w, chunk=chunk)
    return pl.kernel(
        body,
        out_type=jax.ShapeDtypeStruct((rows_total, D), jnp.float32),
        mesh=mesh,
        compiler_params=pltpu.CompilerParams(use_tc_tiling_on_sc=False),
        scratch_types=[
            pltpu.VMEM((chunk,), jnp.int32),
            pltpu.VMEM((chunk, D), jnp.float32),
            pltpu.SemaphoreType.DMA,
        ],
    )


def kernel(state, weight):
    B, N = weight.shape
    D = state.shape[-1]
    JC = 128
    SBLK = 64
    ne = _make_neg_exp(B, N, JC)(weight.reshape(B, N // JC, JC))
    gidx = _make_sampler(B, N, SBLK, JC)(ne).reshape(-1)
    table = state.reshape(B * N, D)
    new_state = _make_gather(B * N, D)(table, gidx).reshape(B, N, D)
    out_weight = jnp.full((B, N), -math.log(N), dtype=weight.dtype)
    return (new_state, out_weight)
